# Initial kernel scaffold; baseline (speedup 1.0000x reference)
import jax
import jax.numpy as jnp
from jax.experimental import pallas as pl


def kernel(hidden_states, hidden_states_unmodulated, timestep, gate_w, W1, W2, sw1, sw2):
    raise NotImplementedError("write your pallas kernel here")


# speedup vs baseline: 1.1756x; 1.1756x over previous
import functools
import math

import jax
import jax.numpy as jnp
from jax.experimental import pallas as pl
from jax.experimental.pallas import tpu as pltpu


def _scores_body(ts_ref, hsu_ref, gw_ref, o_ref):
    dim = ts_ref.shape[-1]
    b = pl.program_id(0)
    ts = ts_ref[pl.ds(b, 1), :]
    hsu = hsu_ref[0]
    logits = jnp.dot(ts, gw_ref[:dim, :], preferred_element_type=jnp.float32)
    logits = logits + jnp.dot(hsu, gw_ref[dim:, :], preferred_element_type=jnp.float32)
    o_ref[0] = jax.nn.sigmoid(logits)


def _shared_ffn_body(x_ref, w1_ref, w2_ref, o_ref):
    x = x_ref[...]
    h = jnp.dot(x, w1_ref[...], preferred_element_type=jnp.float32)
    inner = h.shape[-1] // 2
    a = h[:, :inner]
    b = h[:, inner:]
    g = a * (b * jax.nn.sigmoid(b))
    o_ref[...] = jnp.dot(g, w2_ref[...], preferred_element_type=jnp.float32)


def _expert_ffn_body(x_ref, w1_ref, w2_ref, gate_ref, o_ref):
    x = x_ref[0]
    h = jnp.dot(x, w1_ref[0], preferred_element_type=jnp.float32)
    inner = h.shape[-1] // 2
    a = h[:, :inner]
    b = h[:, inner:]
    g = a * (b * jax.nn.sigmoid(b))
    y = jnp.dot(g, w2_ref[0], preferred_element_type=jnp.float32)
    o_ref[0] = y * gate_ref[0, 0][:, None]


def kernel(hidden_states, hidden_states_unmodulated, timestep, gate_w, W1, W2, sw1, sw2):
    bs, slen, dim = hidden_states.shape
    E = gate_w.shape[1]
    inner = W2.shape[1]
    cap = max(1, math.ceil(slen / E))
    n_tok = bs * slen
    tpe = bs * cap

    BT = 512
    scores = pl.pallas_call(
        _scores_body,
        grid=(bs, slen // BT),
        in_specs=[
            pl.BlockSpec((bs, dim), lambda b, t: (0, 0)),
            pl.BlockSpec((1, BT, dim), lambda b, t: (b, t, 0)),
            pl.BlockSpec((2 * dim, E), lambda b, t: (0, 0)),
        ],
        out_specs=pl.BlockSpec((1, BT, E), lambda b, t: (b, t, 0)),
        out_shape=jax.ShapeDtypeStruct((bs, slen, E), jnp.float32),
    )(timestep, hidden_states_unmodulated, gate_w)

    affinity = jnp.transpose(scores, (0, 2, 1))
    gating, top_idx = jax.lax.top_k(affinity, cap)
    boff = (jnp.arange(bs, dtype=jnp.int32) * slen).reshape(bs, 1, 1)
    gti = jnp.transpose(boff + top_idx, (1, 0, 2)).reshape(-1)
    gflat = jnp.transpose(gating, (1, 0, 2)).reshape(-1)
    sums = jnp.zeros(n_tok, dtype=jnp.float32).at[gti].add(gflat)
    gflat = gflat / (sums[gti] + 1e-12)

    x_flat = hidden_states.reshape(n_tok, dim)
    BT2 = 512
    shared_out = pl.pallas_call(
        _shared_ffn_body,
        grid=(n_tok // BT2,),
        in_specs=[
            pl.BlockSpec((BT2, dim), lambda i: (i, 0)),
            pl.BlockSpec((dim, 2 * inner), lambda i: (0, 0)),
            pl.BlockSpec((inner, dim), lambda i: (0, 0)),
        ],
        out_specs=pl.BlockSpec((BT2, dim), lambda i: (i, 0)),
        out_shape=jax.ShapeDtypeStruct((n_tok, dim), jnp.float32),
    )(x_flat, sw1, sw2)

    ri = x_flat[gti].reshape(E, tpe, dim)
    gmat = gflat.reshape(E, 1, tpe)
    routed = pl.pallas_call(
        _expert_ffn_body,
        grid=(E,),
        in_specs=[
            pl.BlockSpec((1, tpe, dim), lambda e: (e, 0, 0)),
            pl.BlockSpec((1, dim, 2 * inner), lambda e: (e, 0, 0)),
            pl.BlockSpec((1, inner, dim), lambda e: (e, 0, 0)),
            pl.BlockSpec((1, 1, tpe), lambda e: (e, 0, 0)),
        ],
        out_specs=pl.BlockSpec((1, tpe, dim), lambda e: (e, 0, 0)),
        out_shape=jax.ShapeDtypeStruct((E, tpe, dim), jnp.float32),
    )(ri, W1, W2, gmat)

    out = shared_out.at[gti].add(routed.reshape(E * tpe, dim))
    return out.reshape(bs, slen, dim)
